# Initial kernel scaffold; baseline (speedup 1.0000x reference)
import jax
import jax.numpy as jnp
from jax.experimental import pallas as pl


def kernel(x_trace, x_event, ei_follows, ei_belongs, ei_contains, trace_batch, Wp_trace, Wp_event, Wl_ff, bl_ff, Wr_ff, Wl_bt, bl_bt, Wr_bt, Wl_ce, bl_ce, Wr_ce, Wlin_trace, blin_trace, Wlin_event, blin_event, Wact, bact, Wtime, btime, Wrem, brem):
    raise NotImplementedError("write your pallas kernel here")


# speedup vs baseline: 5.9800x; 5.9800x over previous
import functools

import jax
import jax.numpy as jnp
from jax import lax
from jax.experimental import pallas as pl
from jax.experimental.pallas import tpu as pltpu
from jax.experimental.pallas import tpu_sc as plsc

H = 512
N_T = 10000
N_E = 10000
E = 160000
NUM_GRAPHS = 64
NUM_CLASSES = 32

NW = 32
EPW = E // NW
K = 40
NCH = EPW // K
N_PAD = 10240
RPT = N_PAD // 16

R_BLK = 1000
N_BLK = N_T // R_BLK


def _sc_body(xt2, xe, sf0, sf1, sbt, dff, dbt, zrow, zcnt, ones_h,
             off0, off1, obt, ocf, ocb,
             acc, cntf, cntb, sidx, didx, rows, ones_v, sem):
  c = lax.axis_index("c")
  s = lax.axis_index("s")
  wid = s * 2 + c
  base = s * RPT

  def do_pass(x_hbm, src_hbm, dst_hbm, cnt_ref):
    pltpu.sync_copy(src_hbm.at[wid], sidx)
    pltpu.sync_copy(dst_hbm.at[wid], didx)

    def chunk(j, carry):
      pltpu.async_copy(x_hbm.at[sidx.at[j]], rows, sem).wait()
      pltpu.sync_copy(rows, acc.at[didx.at[j]], add=True)
      if cnt_ref is not None:
        pltpu.sync_copy(ones_v, cnt_ref.at[didx.at[j]], add=True)
      return carry

    lax.fori_loop(0, NCH, chunk, 0)

  def zero_acc():
    pltpu.sync_copy(zrow.at[pl.ds(base, RPT)], acc.at[pl.ds(base, RPT)])

  zero_acc()
  pltpu.sync_copy(zcnt.at[pl.ds(base, RPT)], cntf.at[pl.ds(base, RPT)])
  pltpu.sync_copy(zcnt.at[pl.ds(base, RPT)], cntb.at[pl.ds(base, RPT)])
  pltpu.sync_copy(ones_h, ones_v)
  plsc.subcore_barrier()

  do_pass(xt2, sf0, dff, cntf)
  plsc.subcore_barrier()
  pltpu.sync_copy(acc.at[pl.ds(base, RPT)], off0.at[c, pl.ds(base, RPT)])
  zero_acc()
  plsc.subcore_barrier()

  do_pass(xt2, sf1, dff, None)
  plsc.subcore_barrier()
  pltpu.sync_copy(acc.at[pl.ds(base, RPT)], off1.at[c, pl.ds(base, RPT)])
  zero_acc()
  plsc.subcore_barrier()

  do_pass(xe, sbt, dbt, cntb)
  plsc.subcore_barrier()
  pltpu.sync_copy(acc.at[pl.ds(base, RPT)], obt.at[c, pl.ds(base, RPT)])
  pltpu.sync_copy(cntf.at[pl.ds(base, RPT)], ocf.at[c, pl.ds(base, RPT)])
  pltpu.sync_copy(cntb.at[pl.ds(base, RPT)], ocb.at[c, pl.ds(base, RPT)])


def _sc_aggregate(xt2, xe, sf0, sf1, sbt, dff, dbt, zrow, zcnt, ones_h):
  mesh = plsc.VectorSubcoreMesh(core_axis_name="c", subcore_axis_name="s")
  f32 = jnp.float32
  return pl.kernel(
      _sc_body,
      out_type=(
          jax.ShapeDtypeStruct((2, N_PAD, 128), f32),
          jax.ShapeDtypeStruct((2, N_PAD, 128), f32),
          jax.ShapeDtypeStruct((2, N_PAD, 128), f32),
          jax.ShapeDtypeStruct((2, N_PAD, 16), f32),
          jax.ShapeDtypeStruct((2, N_PAD, 16), f32),
      ),
      mesh=mesh,
      scratch_types=[
          pltpu.VMEM_SHARED((N_PAD, 128), f32),
          pltpu.VMEM_SHARED((N_PAD, 16), f32),
          pltpu.VMEM_SHARED((N_PAD, 16), f32),
          pltpu.VMEM((NCH, K), jnp.int32),
          pltpu.VMEM((NCH, K), jnp.int32),
          pltpu.VMEM((K, 128), f32),
          pltpu.VMEM((K, 16), f32),
          pltpu.SemaphoreType.DMA,
      ],
      compiler_params=pltpu.CompilerParams(use_tc_tiling_on_sc=False),
  )(xt2, xe, sf0, sf1, sbt, dff, dbt, zrow, zcnt, ones_h)


def _tc_body(aff0, aff1, abt, cff, cbt, xt, tb,
             wpt, wpe, wlff, wlbt, wrff, wrbt, blff, blbt,
             wlin, blin, wh, bh,
             head_o,
             A_ff, A_bt, A_r, psum, pcnt):
  i = pl.program_id(0)
  f32 = jnp.float32
  dn = (((0,), (1,)), ((), ()))

  @pl.when(i == 0)
  def _():
    A_ff[...] = lax.dot_general(wpt[...], wlff[...], dn,
                                preferred_element_type=f32,
                                precision=lax.Precision.HIGHEST)
    A_bt[...] = lax.dot_general(wpe[...], wlbt[...], dn,
                                preferred_element_type=f32,
                                precision=lax.Precision.HIGHEST)
    A_r[...] = lax.dot_general(wpt[...], wrff[...] + wrbt[...], dn,
                               preferred_element_type=f32,
                                precision=lax.Precision.HIGHEST)
    psum[...] = jnp.zeros_like(psum)
    pcnt[...] = jnp.zeros_like(pcnt)

  nff = jnp.maximum(cff[0, :, 0] + cff[1, :, 0], 1.0)
  nbt = jnp.maximum(cbt[0, :, 0] + cbt[1, :, 0], 1.0)
  mff = jnp.concatenate([aff0[0] + aff0[1], aff1[0] + aff1[1]], axis=1)
  mff = mff / nff[:, None]
  mbt = (abt[0] + abt[1]) / nbt[:, None]

  dnm = (((1,), (0,)), ((), ()))
  o = (lax.dot_general(mff, A_ff[...], dnm, preferred_element_type=f32,
                                precision=lax.Precision.HIGHEST)
       + lax.dot_general(mbt, A_bt[...], dnm, preferred_element_type=f32,
                                precision=lax.Precision.HIGHEST)
       + lax.dot_general(xt[...], A_r[...], dnm, preferred_element_type=f32,
                                precision=lax.Precision.HIGHEST)
       + (blff[...] + blbt[...])[None, :])
  dnt = (((1,), (1,)), ((), ()))
  t = jnp.maximum(
      lax.dot_general(o, wlin[...], dnt, preferred_element_type=f32,
                                precision=lax.Precision.HIGHEST)
      + blin[...][None, :], 0.0)

  ids = tb[0, 0, :]
  p = (ids[:, None] == lax.broadcasted_iota(jnp.int32, (R_BLK, NUM_GRAPHS),
                                            1)).astype(f32)
  psum[...] += lax.dot_general(p, t, (((0,), (0,)), ((), ())),
                               preferred_element_type=f32,
                                precision=lax.Precision.HIGHEST)
  pcnt[...] += jnp.sum(p, axis=0)

  @pl.when(i == N_BLK - 1)
  def _():
    pooled = psum[...] / jnp.maximum(pcnt[...], 1.0)[:, None]
    head_o[...] = (lax.dot_general(pooled, wh[...], dnt,
                                   preferred_element_type=f32,
                                precision=lax.Precision.HIGHEST)
                   + bh[...][None, :])


def _tc_dense(aff0, aff1, abt, cff, cbt, xt, tb3, wpt, wpe,
              wlff, wlbt, wrff, wrbt, blff, blbt, wlin, blin, wh, bh):
  f32 = jnp.float32
  full = lambda shp: pl.BlockSpec(shp, lambda i: tuple(0 for _ in shp))
  grid_spec = pltpu.PrefetchScalarGridSpec(
      num_scalar_prefetch=0,
      grid=(N_BLK,),
      in_specs=[
          pl.BlockSpec((2, R_BLK, 128), lambda i: (0, i, 0)),
          pl.BlockSpec((2, R_BLK, 128), lambda i: (0, i, 0)),
          pl.BlockSpec((2, R_BLK, 128), lambda i: (0, i, 0)),
          pl.BlockSpec((2, R_BLK, 16), lambda i: (0, i, 0)),
          pl.BlockSpec((2, R_BLK, 16), lambda i: (0, i, 0)),
          pl.BlockSpec((R_BLK, 256), lambda i: (i, 0)),
          pl.BlockSpec((1, 1, R_BLK), lambda i: (i, 0, 0)),
          full((H, 256)), full((H, 128)),
          full((H, H)), full((H, H)), full((H, H)), full((H, H)),
          full((H,)), full((H,)),
          full((H, H)), full((H,)),
          full((NUM_GRAPHS, H)), full((NUM_GRAPHS,)),
      ],
      out_specs=[
          pl.BlockSpec((NUM_GRAPHS, NUM_GRAPHS), lambda i: (0, 0)),
      ],
      scratch_shapes=[
          pltpu.VMEM((256, H), f32),
          pltpu.VMEM((128, H), f32),
          pltpu.VMEM((256, H), f32),
          pltpu.VMEM((NUM_GRAPHS, H), f32),
          pltpu.VMEM((NUM_GRAPHS,), f32),
      ],
  )
  return pl.pallas_call(
      _tc_body,
      grid_spec=grid_spec,
      out_shape=(jax.ShapeDtypeStruct((NUM_GRAPHS, NUM_GRAPHS), f32),),
  )(aff0, aff1, abt, cff, cbt, xt, tb3, wpt, wpe, wlff, wlbt, wrff, wrbt,
    blff, blbt, wlin, blin, wh, bh)[0]


def kernel(x_trace, x_event, ei_follows, ei_belongs, ei_contains,
           trace_batch, Wp_trace, Wp_event,
           Wl_ff, bl_ff, Wr_ff, Wl_bt, bl_bt, Wr_bt, Wl_ce, bl_ce, Wr_ce,
           Wlin_trace, blin_trace, Wlin_event, blin_event,
           Wact, bact, Wtime, btime, Wrem, brem):
  i32 = jnp.int32
  f32 = jnp.float32
  src_ff = ei_follows[0].astype(i32)
  dst_ff = ei_follows[1].astype(i32).reshape(NW, NCH, K)
  src_bt = ei_belongs[0].astype(i32).reshape(NW, NCH, K)
  dst_bt = ei_belongs[1].astype(i32).reshape(NW, NCH, K)
  sf0 = (src_ff * 2).reshape(NW, NCH, K)
  sf1 = (src_ff * 2 + 1).reshape(NW, NCH, K)
  xt2 = x_trace.reshape(2 * N_T, 128)
  zrow = jnp.zeros((N_PAD, 128), f32)
  zcnt = jnp.zeros((N_PAD, 16), f32)
  ones_h = jnp.ones((K, 16), f32)

  aff0, aff1, abt, cff, cbt = _sc_aggregate(
      xt2, x_event, sf0, sf1, src_bt, dst_ff, dst_bt, zrow, zcnt, ones_h)

  tb3 = trace_batch.astype(i32).reshape(N_BLK, 1, R_BLK)
  npad = NUM_GRAPHS - NUM_CLASSES - 2
  wh = jnp.concatenate([Wact, Wtime, Wrem, jnp.zeros((npad, H), f32)], axis=0)
  bh = jnp.concatenate([bact, btime, brem, jnp.zeros((npad,), f32)])
  hout = _tc_dense(
      aff0, aff1, abt, cff, cbt, x_trace, tb3, Wp_trace, Wp_event,
      Wl_ff, Wl_bt, Wr_ff, Wr_bt, bl_ff, bl_bt, Wlin_trace, blin_trace,
      wh, bh)
  return (hout[:, :NUM_CLASSES], hout[:, NUM_CLASSES],
          hout[:, NUM_CLASSES + 1])
